# baseline (speedup 1.0000x reference)
import functools

import jax
import jax.numpy as jnp
from jax import lax
from jax.experimental import pallas as pl
from jax.experimental.pallas import tpu as pltpu
from jax.experimental.pallas import tpu_sc as plsc

N = 10000
E = 320000
D = 128
DA = 144
NC = 2
NS = 16
NW = NC * NS
K = 128
CPW = 80
NCH = NW * (CPW + 2)
EP = NCH * K
NPAD = N + 16
RPT = 624
RTAIL = N - NS * RPT
DW = 16


def _sc_body(want_deg, din, *refs):
    if want_deg:
        (x_hbm, src_hbm, dst_hbm, z2d, agg_out, deg_out, agg_sh,
         rows0, rows1, sidx0, sidx1, didx0, didx1, *sems) = refs
    else:
        (x_hbm, src_hbm, dst_hbm, z2d, agg_out, agg_sh,
         rows0, rows1, sidx0, sidx1, didx0, didx1, *sems) = refs
        deg_out = None
    rows = (rows0, rows1)
    sidx = (sidx0, sidx1)
    didx = (didx0, didx1)
    sem_g = sems[0:2]
    sem_s = sems[2:4]
    sem_is = sems[4:6]
    sem_id = sems[6:8]

    cid = lax.axis_index("c")
    sid = lax.axis_index("s")
    wid = cid * NS + sid

    pltpu.sync_copy(z2d.at[pl.ds(sid * RPT, RPT)], agg_sh.at[pl.ds(sid * RPT, RPT)])

    @pl.when(sid == NS - 1)
    def _():
        t0 = pl.ds(NS * RPT, NPAD - NS * RPT)
        pltpu.sync_copy(z2d.at[t0], agg_sh.at[t0])

    for s in (0, 1):
        c = s * NW + wid
        pltpu.sync_copy(src_hbm.at[c], sidx[s])
        pltpu.async_copy(x_hbm.at[sidx[s]], rows[s], sem_g[s])
        pltpu.async_copy(dst_hbm.at[c], didx[s], sem_id[s])

    plsc.subcore_barrier()

    def step(i, _):
        for s in (0, 1):
            j = 2 * i + s
            pltpu.make_async_copy(x_hbm.at[pl.ds(0, K)], rows[s], sem_g[s]).wait()
            pltpu.async_copy(src_hbm.at[(j + 2) * NW + wid], sidx[s], sem_is[s])
            pltpu.make_async_copy(dst_hbm.at[0], didx[s], sem_id[s]).wait()
            dsc = pltpu.async_copy(rows[s], agg_sh.at[didx[s]], sem_s[s],
                                   add=True)
            dsc.wait()
            pltpu.async_copy(dst_hbm.at[(j + 2) * NW + wid], didx[s], sem_id[s])
            pltpu.make_async_copy(src_hbm.at[0], sidx[s], sem_is[s]).wait()
            pltpu.async_copy(x_hbm.at[sidx[s]], rows[s], sem_g[s])
        return 0

    lax.fori_loop(0, CPW // 2, step, 0)
    for s in (0, 1):
        pltpu.make_async_copy(x_hbm.at[pl.ds(0, K)], rows[s], sem_g[s]).wait()
        pltpu.make_async_copy(dst_hbm.at[0], didx[s], sem_id[s]).wait()
    plsc.subcore_barrier()

    r0 = pl.ds(sid * RPT, RPT)
    o0 = pl.ds(cid * N + sid * RPT, RPT)
    pltpu.sync_copy(agg_sh.at[r0, pl.ds(0, D)], agg_out.at[o0])
    if want_deg:
        pltpu.sync_copy(agg_sh.at[r0, pl.ds(D, DW)], deg_out.at[o0])

    @pl.when(sid == NS - 1)
    def _():
        t0 = pl.ds(NS * RPT, RTAIL)
        to = pl.ds(cid * N + NS * RPT, RTAIL)
        pltpu.sync_copy(agg_sh.at[t0, pl.ds(0, D)], agg_out.at[to])
        if want_deg:
            pltpu.sync_copy(agg_sh.at[t0, pl.ds(D, DW)], deg_out.at[to])


def _make_sc(want_deg):
    din = DA if want_deg else D
    mesh = plsc.VectorSubcoreMesh(core_axis_name="c", subcore_axis_name="s")
    out_type = [jax.ShapeDtypeStruct((NC * N, D), jnp.float32)]
    if want_deg:
        out_type.append(jax.ShapeDtypeStruct((NC * N, DW), jnp.float32))
    shared = [pltpu.VMEM_SHARED((NPAD, din), jnp.float32)]
    pipe = ([pltpu.VMEM((K, din), jnp.float32)] * 2
            + [pltpu.VMEM((K,), jnp.int32)] * 4
            + [pltpu.SemaphoreType.DMA] * 8)
    return pl.kernel(
        functools.partial(_sc_body, want_deg, din),
        out_type=tuple(out_type) if want_deg else out_type[0],
        mesh=mesh,
        scratch_types=shared + pipe,
        compiler_params=pltpu.CompilerParams(use_tc_tiling_on_sc=False),
    )


_sc_agg_deg = _make_sc(True)
_sc_agg = _make_sc(False)


def _tc_layer(x, aggp, degp, w_self, w_neigh, b, relu):
    nb = 10
    br = N // nb

    def body(x_ref, aggp_ref, degp_ref, ws_ref, wn_ref, b_ref, o_ref):
        agg = aggp_ref[0] + aggp_ref[1]
        deg = degp_ref[0, :, 0:1] + degp_ref[1, :, 0:1]
        h = agg / jnp.maximum(deg, 1.0)
        o = (jnp.dot(x_ref[...], ws_ref[...], preferred_element_type=jnp.float32)
             + jnp.dot(h, wn_ref[...], preferred_element_type=jnp.float32)
             + b_ref[...])
        o_ref[...] = jnp.maximum(o, 0.0) if relu else o

    return pl.pallas_call(
        body,
        grid=(nb,),
        in_specs=[
            pl.BlockSpec((br, D), lambda i: (i, 0)),
            pl.BlockSpec((2, br, D), lambda i: (0, i, 0)),
            pl.BlockSpec((2, br, DW), lambda i: (0, i, 0)),
            pl.BlockSpec((D, D), lambda i: (0, 0)),
            pl.BlockSpec((D, D), lambda i: (0, 0)),
            pl.BlockSpec((1, D), lambda i: (0, 0)),
        ],
        out_specs=pl.BlockSpec((br, D), lambda i: (i, 0)),
        out_shape=jax.ShapeDtypeStruct((N, D), jnp.float32),
    )(x, aggp.reshape(2, N, D), degp.reshape(2, N, DW), w_self, w_neigh,
      b.reshape(1, D))


def kernel(x, edge_index, W_self0, W_neigh0, b0, W_self1, W_neigh1, b1):
    src = edge_index[0].astype(jnp.int32)
    dst = edge_index[1].astype(jnp.int32)
    pad = EP - E
    src2d = jnp.concatenate([src, jnp.zeros((pad,), jnp.int32)]).reshape(NCH, K)
    dst2d = jnp.concatenate([dst, jnp.full((pad,), N, jnp.int32)]).reshape(NCH, K)
    x_aug = jnp.concatenate(
        [x, jnp.ones((N, 1), jnp.float32), jnp.zeros((N, DA - D - 1), jnp.float32)],
        axis=1)
    zA = jnp.zeros((NPAD, DA), jnp.float32)
    zD = jnp.zeros((NPAD, D), jnp.float32)

    aggp0, degp = _sc_agg_deg(x_aug, src2d, dst2d, zA)
    h = _tc_layer(x, aggp0, degp, W_self0, W_neigh0, b0, relu=True)
    aggp1 = _sc_agg(h, src2d, dst2d, zD)
    return _tc_layer(h, aggp1, degp, W_self1, W_neigh1, b1, relu=False)


# speedup vs baseline: 1.9189x; 1.9189x over previous
import functools

import jax
import jax.numpy as jnp
from jax import lax
from jax.experimental import pallas as pl
from jax.experimental.pallas import tpu as pltpu
from jax.experimental.pallas import tpu_sc as plsc

N = 10000
E = 320000
D = 128
DA = 144
NC = 2
NS = 16
NW = NC * NS
K = 128
CPW = 79
NCH = NW * CPW
EP = NCH * K
NPAD = N + 16
RPT = 624
RTAIL = N - NS * RPT
DW = 16


def _sc_body(want_deg, din, *refs):
    if want_deg:
        (x_hbm, eidx_hbm, z2d, agg_out, deg_out, agg_sh, rows_v, eidx_v) = refs
    else:
        (x_hbm, eidx_hbm, z2d, agg_out, agg_sh, rows_v, eidx_v) = refs
        deg_out = None

    cid = lax.axis_index("c")
    sid = lax.axis_index("s")
    wid = cid * NS + sid

    pltpu.sync_copy(z2d.at[pl.ds(sid * RPT, RPT)], agg_sh.at[pl.ds(sid * RPT, RPT)])

    @pl.when(sid == NS - 1)
    def _():
        t0 = pl.ds(NS * RPT, NPAD - NS * RPT)
        pltpu.sync_copy(z2d.at[t0], agg_sh.at[t0])

    plsc.subcore_barrier()

    def step(j, _):
        c = j * NW + wid
        pltpu.sync_copy(eidx_hbm.at[c], eidx_v)
        pltpu.sync_copy(x_hbm.at[eidx_v.at[0]], rows_v)
        pltpu.sync_copy(rows_v, agg_sh.at[eidx_v.at[1]], add=True)
        return 0

    lax.fori_loop(0, CPW, step, 0)
    plsc.subcore_barrier()

    r0 = pl.ds(sid * RPT, RPT)
    o0 = pl.ds(cid * N + sid * RPT, RPT)
    pltpu.sync_copy(agg_sh.at[r0, pl.ds(0, D)], agg_out.at[o0])
    if want_deg:
        pltpu.sync_copy(agg_sh.at[r0, pl.ds(D, DW)], deg_out.at[o0])

    @pl.when(sid == NS - 1)
    def _():
        t0 = pl.ds(NS * RPT, RTAIL)
        to = pl.ds(cid * N + NS * RPT, RTAIL)
        pltpu.sync_copy(agg_sh.at[t0, pl.ds(0, D)], agg_out.at[to])
        if want_deg:
            pltpu.sync_copy(agg_sh.at[t0, pl.ds(D, DW)], deg_out.at[to])


def _make_sc(want_deg):
    din = DA if want_deg else D
    mesh = plsc.VectorSubcoreMesh(core_axis_name="c", subcore_axis_name="s")
    out_type = [jax.ShapeDtypeStruct((NC * N, D), jnp.float32)]
    if want_deg:
        out_type.append(jax.ShapeDtypeStruct((NC * N, DW), jnp.float32))
    scratch = [
        pltpu.VMEM_SHARED((NPAD, din), jnp.float32),
        pltpu.VMEM((K, din), jnp.float32),
        pltpu.VMEM((2, K), jnp.int32),
    ]
    return pl.kernel(
        functools.partial(_sc_body, want_deg, din),
        out_type=tuple(out_type) if want_deg else out_type[0],
        mesh=mesh,
        scratch_types=scratch,
        compiler_params=pltpu.CompilerParams(use_tc_tiling_on_sc=False),
    )


_sc_agg_deg = _make_sc(True)
_sc_agg = _make_sc(False)


def _tc_layer(x, aggp, degp, w_self, w_neigh, b, relu):
    nb = 10
    br = N // nb

    def body(x_ref, aggp_ref, degp_ref, ws_ref, wn_ref, b_ref, o_ref):
        agg = aggp_ref[0] + aggp_ref[1]
        deg = degp_ref[0, :, 0:1] + degp_ref[1, :, 0:1]
        h = agg / jnp.maximum(deg, 1.0)
        o = (jnp.dot(x_ref[...], ws_ref[...], preferred_element_type=jnp.float32)
             + jnp.dot(h, wn_ref[...], preferred_element_type=jnp.float32)
             + b_ref[...])
        o_ref[...] = jnp.maximum(o, 0.0) if relu else o

    return pl.pallas_call(
        body,
        grid=(nb,),
        in_specs=[
            pl.BlockSpec((br, D), lambda i: (i, 0)),
            pl.BlockSpec((2, br, D), lambda i: (0, i, 0)),
            pl.BlockSpec((2, br, DW), lambda i: (0, i, 0)),
            pl.BlockSpec((D, D), lambda i: (0, 0)),
            pl.BlockSpec((D, D), lambda i: (0, 0)),
            pl.BlockSpec((1, D), lambda i: (0, 0)),
        ],
        out_specs=pl.BlockSpec((br, D), lambda i: (i, 0)),
        out_shape=jax.ShapeDtypeStruct((N, D), jnp.float32),
    )(x, aggp.reshape(2, N, D), degp.reshape(2, N, DW), w_self, w_neigh,
      b.reshape(1, D))


def kernel(x, edge_index, W_self0, W_neigh0, b0, W_self1, W_neigh1, b1):
    src = edge_index[0].astype(jnp.int32)
    dst = edge_index[1].astype(jnp.int32)
    pad = EP - E
    src2d = jnp.concatenate([src, jnp.zeros((pad,), jnp.int32)]).reshape(NCH, 1, K)
    dst2d = jnp.concatenate([dst, jnp.full((pad,), N, jnp.int32)]).reshape(NCH, 1, K)
    eidx = jnp.concatenate([src2d, dst2d], axis=1)
    x_aug = jnp.concatenate(
        [x, jnp.ones((N, 1), jnp.float32), jnp.zeros((N, DA - D - 1), jnp.float32)],
        axis=1)
    zA = jnp.zeros((NPAD, DA), jnp.float32)
    zD = jnp.zeros((NPAD, D), jnp.float32)

    aggp0, degp = _sc_agg_deg(x_aug, eidx, zA)
    h = _tc_layer(x, aggp0, degp, W_self0, W_neigh0, b0, relu=True)
    aggp1 = _sc_agg(h, eidx, zD)
    return _tc_layer(h, aggp1, degp, W_self1, W_neigh1, b1, relu=False)
